# baseline (speedup 1.0000x reference)
import functools

import jax
import jax.numpy as jnp
from jax import lax
from jax.experimental import pallas as pl
from jax.experimental.pallas import tpu as pltpu
from jax.experimental.pallas import tpu_sc as plsc

D = 64
NW = 32
HALF = 100
CHUNK = 200
NBUF = 2


@jax.jit
def kernel(x, token_table, pos_table):
    B, S = x.shape
    rows = B * S
    assert S == CHUNK and D == token_table.shape[1]
    rpw = rows // NW
    nchunk = rpw // CHUNK
    assert rpw * NW == rows and nchunk * CHUNK == rpw and nchunk % NBUF == 0

    idx = x.astype(jnp.int32).reshape(NW, nchunk, 2, HALF)

    mesh = plsc.VectorSubcoreMesh(core_axis_name="c", subcore_axis_name="s")

    @functools.partial(
        pl.kernel,
        out_type=jax.ShapeDtypeStruct((rows, D), jnp.float32),
        mesh=mesh,
        compiler_params=pltpu.CompilerParams(use_tc_tiling_on_sc=False),
        scratch_types=[
            pltpu.VMEM((nchunk, 2, HALF), jnp.int32),
            pltpu.VMEM_SHARED((S, D), jnp.float32),
            pltpu.VMEM((NBUF, CHUNK, D), jnp.float32),
            pltpu.SemaphoreType.DMA,
            pltpu.SemaphoreType.DMA,
            pltpu.SemaphoreType.DMA,
            pltpu.SemaphoreType.DMA,
            pltpu.SemaphoreType.DMA,
        ],
    )
    def embed(idx_hbm, tok_hbm, pos_hbm, out_hbm,
              idx_v, pos_sh, rows_v, isem0, isem1, gsem, osem0, osem1):
        wid = lax.axis_index("s") * 2 + lax.axis_index("c")
        base = wid * rpw
        isems = (isem0, isem1)
        osems = (osem0, osem1)

        @pl.when(lax.axis_index("s") == 0)
        def _():
            pltpu.sync_copy(pos_hbm, pos_sh)

        pltpu.sync_copy(idx_hbm.at[wid], idx_v)
        plsc.subcore_barrier()

        pltpu.async_copy(pos_sh, rows_v.at[0], isems[0])

        def out_slice(j):
            return out_hbm.at[pl.ds(base + j * CHUNK, CHUNK)]

        def step(j, slot):
            buf = rows_v.at[slot]
            nslot = (slot + 1) % NBUF
            nbuf = rows_v.at[nslot]
            pltpu.make_async_copy(pos_sh, buf, isems[slot]).wait()
            c0 = pltpu.async_copy(tok_hbm.at[idx_v.at[j, 0]],
                                  buf.at[pl.ds(0, HALF)], gsem, add=True)
            c1 = pltpu.async_copy(tok_hbm.at[idx_v.at[j, 1]],
                                  buf.at[pl.ds(HALF, HALF)], gsem, add=True)

            @pl.when(j + 1 < nchunk)
            def _():
                @pl.when(j + 1 >= NBUF)
                def _():
                    pltpu.make_async_copy(nbuf, out_slice(j), osems[nslot]).wait()
                pltpu.async_copy(pos_sh, nbuf, isems[nslot])

            c0.wait()
            c1.wait()
            pltpu.async_copy(buf, out_slice(j), osems[slot])

        def outer(g, carry):
            for b in range(NBUF):
                step(g * NBUF + b, b)
            return carry

        lax.fori_loop(0, nchunk // NBUF, outer, 0)
        last = (nchunk - 1) % NBUF
        pltpu.make_async_copy(
            rows_v.at[last], out_slice(nchunk - 1), osems[last]
        ).wait()

    out = embed(idx, token_table, pos_table)
    return out.reshape(B, S, D)


# speedup vs baseline: 1.0033x; 1.0033x over previous
import functools

import jax
import jax.numpy as jnp
from jax import lax
from jax.experimental import pallas as pl
from jax.experimental.pallas import tpu as pltpu
from jax.experimental.pallas import tpu_sc as plsc

D = 64
NW = 32
HALF_A = 104
HALF_B = 96
NBUF = 2


@jax.jit
def kernel(x, token_table, pos_table):
    B, S = x.shape
    assert S == HALF_A + HALF_B and D == token_table.shape[1]
    spw = B // NW
    assert spw * NW == B and spw % NBUF == 0

    mesh = plsc.VectorSubcoreMesh(core_axis_name="c", subcore_axis_name="s")

    @functools.partial(
        pl.kernel,
        out_type=jax.ShapeDtypeStruct((B, S, D), jnp.float32),
        mesh=mesh,
        compiler_params=pltpu.CompilerParams(use_tc_tiling_on_sc=False),
        scratch_types=[
            pltpu.VMEM((spw, S), jnp.int32),
            pltpu.VMEM_SHARED((S, D), jnp.float32),
            pltpu.VMEM((NBUF, S, D), jnp.float32),
            pltpu.SemaphoreType.DMA,
            pltpu.SemaphoreType.DMA,
            pltpu.SemaphoreType.DMA,
            pltpu.SemaphoreType.DMA,
            pltpu.SemaphoreType.DMA,
        ],
    )
    def embed(x_hbm, tok_hbm, pos_hbm, out_hbm,
              idx_v, pos_sh, rows_v, isem0, isem1, gsem, osem0, osem1):
        wid = lax.axis_index("s") * 2 + lax.axis_index("c")
        seq0 = wid * spw
        isems = (isem0, isem1)
        osems = (osem0, osem1)

        @pl.when(lax.axis_index("s") == 0)
        def _():
            pltpu.sync_copy(pos_hbm, pos_sh)

        pltpu.sync_copy(x_hbm.at[pl.ds(seq0, spw)], idx_v)
        plsc.subcore_barrier()

        pltpu.async_copy(pos_sh, rows_v.at[0], isems[0])

        def step(j, slot):
            buf = rows_v.at[slot]
            nslot = (slot + 1) % NBUF
            nbuf = rows_v.at[nslot]
            pltpu.make_async_copy(pos_sh, buf, isems[slot]).wait()
            c0 = pltpu.async_copy(tok_hbm.at[idx_v.at[j, pl.ds(0, HALF_A)]],
                                  buf.at[pl.ds(0, HALF_A)], gsem, add=True)
            c1 = pltpu.async_copy(tok_hbm.at[idx_v.at[j, pl.ds(HALF_A, HALF_B)]],
                                  buf.at[pl.ds(HALF_A, HALF_B)], gsem, add=True)

            @pl.when(j + 1 < spw)
            def _():
                @pl.when(j + 1 >= NBUF)
                def _():
                    pltpu.make_async_copy(nbuf, out_hbm.at[seq0 + j],
                                          osems[nslot]).wait()
                pltpu.async_copy(pos_sh, nbuf, isems[nslot])

            c0.wait()
            c1.wait()
            pltpu.async_copy(buf, out_hbm.at[seq0 + j], osems[slot])

        def outer(g, carry):
            for b in range(NBUF):
                step(g * NBUF + b, b)
            return carry

        lax.fori_loop(0, spw // NBUF, outer, 0)
        last = (spw - 1) % NBUF
        pltpu.make_async_copy(
            rows_v.at[last], out_hbm.at[seq0 + spw - 1], osems[last]
        ).wait()

    return embed(x, token_table, pos_table)


# speedup vs baseline: 1.3203x; 1.3160x over previous
import functools

import jax
import jax.numpy as jnp
from jax import lax
from jax.experimental import pallas as pl
from jax.experimental.pallas import tpu as pltpu
from jax.experimental.pallas import tpu_sc as plsc

D = 64
DP = 128
NW = 32
HALF_A = 104
HALF_B = 96
NBUF = 2


@jax.jit
def kernel(x, token_table, pos_table):
    B, S = x.shape
    assert S == HALF_A + HALF_B and D == token_table.shape[1]
    spw = B // NW
    assert spw * NW == B and spw % NBUF == 0

    mesh = plsc.VectorSubcoreMesh(core_axis_name="c", subcore_axis_name="s")

    @functools.partial(
        pl.kernel,
        out_type=jax.ShapeDtypeStruct((B, S, DP), jnp.float32),
        mesh=mesh,
        compiler_params=pltpu.CompilerParams(use_tc_tiling_on_sc=False),
        scratch_types=[
            pltpu.VMEM((spw, S), jnp.int32),
            pltpu.VMEM_SHARED((S, D), jnp.float32),
            pltpu.VMEM((NBUF, S, D), jnp.float32),
            pltpu.SemaphoreType.DMA,
            pltpu.SemaphoreType.DMA,
            pltpu.SemaphoreType.DMA,
            pltpu.SemaphoreType.DMA,
            pltpu.SemaphoreType.DMA,
        ],
    )
    def embed(x_hbm, tok_hbm, pos_hbm, out_hbm,
              idx_v, pos_sh, rows_v, isem0, isem1, gsem, osem0, osem1):
        wid = lax.axis_index("s") * 2 + lax.axis_index("c")
        seq0 = wid * spw
        isems = (isem0, isem1)
        osems = (osem0, osem1)

        @pl.when(lax.axis_index("s") == 0)
        def _():
            pltpu.sync_copy(pos_hbm, pos_sh)

        pltpu.sync_copy(x_hbm.at[pl.ds(seq0, spw)], idx_v)
        plsc.subcore_barrier()

        pltpu.async_copy(pos_sh, rows_v.at[0], isems[0])

        def step(j, slot):
            buf = rows_v.at[slot]
            nslot = (slot + 1) % NBUF
            nbuf = rows_v.at[nslot]
            pltpu.make_async_copy(pos_sh, buf, isems[slot]).wait()
            c0 = pltpu.async_copy(tok_hbm.at[idx_v.at[j, pl.ds(0, HALF_A)]],
                                  buf.at[pl.ds(0, HALF_A)], gsem, add=True)
            c1 = pltpu.async_copy(tok_hbm.at[idx_v.at[j, pl.ds(HALF_A, HALF_B)]],
                                  buf.at[pl.ds(HALF_A, HALF_B)], gsem, add=True)

            @pl.when(j + 1 < spw)
            def _():
                @pl.when(j + 1 >= NBUF)
                def _():
                    pltpu.make_async_copy(nbuf, out_hbm.at[seq0 + j, :, pl.ds(0, D)],
                                          osems[nslot]).wait()
                pltpu.async_copy(pos_sh, nbuf, isems[nslot])

            c0.wait()
            c1.wait()
            pltpu.async_copy(buf, out_hbm.at[seq0 + j, :, pl.ds(0, D)],
                             osems[slot])

        def outer(g, carry):
            for b in range(NBUF):
                step(g * NBUF + b, b)
            return carry

        lax.fori_loop(0, spw // NBUF, outer, 0)
        last = (spw - 1) % NBUF
        pltpu.make_async_copy(
            rows_v.at[last], out_hbm.at[seq0 + spw - 1, :, pl.ds(0, D)],
            osems[last]
        ).wait()

    out128 = embed(x, token_table, pos_table)
    return out128[:, :, :D]
